# baseline (speedup 1.0000x reference)
import functools

import jax
import jax.numpy as jnp
from jax.experimental import pallas as pl
from jax.experimental.pallas import tpu as pltpu

_P = 128
_SLOTS = 4
_QI = 4
_QO = 2


def _body(x_hbm, o_hbm, bufi, obuf, insem, outsem, *, n_steps, h2):
    r2 = 2 * _P
    pq = r2 // _QI
    po = _P // _QO

    def dma_in(slot, step):
        r0 = step * r2
        for q in range(_QI):
            pltpu.make_async_copy(
                x_hbm.at[pl.ds(r0 + q * pq, pq), :, :],
                bufi.at[slot, pl.ds(q * pq, pq), :, :],
                insem.at[slot],
            ).start()

    def wait_in(slot):
        pltpu.make_async_copy(
            x_hbm.at[pl.ds(0, r2), :, :],
            bufi.at[slot],
            insem.at[slot],
        ).wait()

    def dma_out(slot, step):
        b = step // (h2 // _P)
        h0 = (step % (h2 // _P)) * _P
        for c in range(4):
            for q in range(_QO):
                pltpu.make_async_copy(
                    obuf.at[slot, c, pl.ds(q * po, po), :, :],
                    o_hbm.at[b, c, pl.ds(h0 + q * po, po), :, :],
                    outsem.at[slot],
                ).start()

    def wait_out(slot):
        pltpu.make_async_copy(
            obuf.at[slot],
            o_hbm.at[0, :, pl.ds(0, _P), :, :],
            outsem.at[slot],
        ).wait()

    def compute(slot):
        i = jax.lax.broadcasted_iota(jnp.int32, (_P, 8, 128), 2)
        perm = jnp.where(i < 64, 2 * i, 2 * i - 127)
        for rp in range(2):
            a = bufi[slot, pl.ds(rp, _P, 2), pl.ds(0, 8, 2), :]
            bb = bufi[slot, pl.ds(rp, _P, 2), pl.ds(1, 8, 2), :]
            ga = jnp.take_along_axis(a, perm, axis=2)
            gb = jnp.take_along_axis(bb, perm, axis=2)
            obuf[slot, 2 * rp] = jnp.concatenate(
                [ga[:, :, :64], gb[:, :, :64]], axis=2)
            obuf[slot, 2 * rp + 1] = jnp.concatenate(
                [ga[:, :, 64:], gb[:, :, 64:]], axis=2)

    for s0 in range(_SLOTS - 1):
        dma_in(s0, s0)

    def step_fn(s, _):
        slot = jax.lax.rem(s, _SLOTS)
        nxt = jax.lax.rem(s + _SLOTS - 1, _SLOTS)

        @pl.when(s + _SLOTS - 1 < n_steps)
        def _():
            dma_in(nxt, s + _SLOTS - 1)

        wait_in(slot)

        @pl.when(s >= _SLOTS)
        def _():
            wait_out(slot)

        compute(slot)
        dma_out(slot, s)
        return ()

    jax.lax.fori_loop(0, n_steps, step_fn, ())
    for s0 in range(_SLOTS):
        wait_out(jax.lax.rem(n_steps - _SLOTS + s0, _SLOTS))


def kernel(x):
    B, C, H, W = x.shape
    H2, W2 = H // 2, W // 2
    G = W // 128
    x2 = x.reshape(B * H, G, 128)
    n_steps = (B * H) // (2 * _P)
    body = functools.partial(_body, n_steps=n_steps, h2=H2)
    out = pl.pallas_call(
        body,
        in_specs=[pl.BlockSpec(memory_space=pltpu.MemorySpace.HBM)],
        out_specs=pl.BlockSpec(memory_space=pltpu.MemorySpace.HBM),
        out_shape=jax.ShapeDtypeStruct((B, 4 * C, H2, G // 2, 128), x.dtype),
        scratch_shapes=[
            pltpu.VMEM((_SLOTS, 2 * _P, G, 128), x.dtype),
            pltpu.VMEM((_SLOTS, 4, _P, G // 2, 128), x.dtype),
            pltpu.SemaphoreType.DMA((_SLOTS,)),
            pltpu.SemaphoreType.DMA((_SLOTS,)),
        ],
    )(x2)
    return out.reshape(B, 4 * C, H2, W2)


# speedup vs baseline: 1.3220x; 1.3220x over previous
import functools

import jax
import jax.numpy as jnp
from jax.experimental import pallas as pl
from jax.experimental.pallas import tpu as pltpu

_P = 128
_SLOTS = 4
_QI = 4
_QO = 2


def _deinterleave(v):
    p, w = v.shape
    i = jax.lax.broadcasted_iota(jnp.int32, (p, 128), 1)
    perm = jnp.where(i < 64, 2 * i, 2 * i - 127)
    ev, od = [], []
    for g in range(w // 128):
        y = jnp.take_along_axis(v[:, g * 128:(g + 1) * 128], perm, axis=1)
        ev.append(y[:, :64])
        od.append(y[:, 64:])
    return jnp.concatenate(ev, axis=1), jnp.concatenate(od, axis=1)


def _body(x_hbm, o_hbm, bufe, bufo, obuf, insem, outsem, *, n_steps, h2):
    pq = _P // _QI
    po = _P // _QO

    def dma_in(slot, step):
        r0 = step * _P
        for q in range(_QI):
            pltpu.make_async_copy(
                x_hbm.at[pl.ds(r0 + q * pq, pq), 0, :],
                bufe.at[slot, pl.ds(q * pq, pq), :],
                insem.at[slot],
            ).start()
            pltpu.make_async_copy(
                x_hbm.at[pl.ds(r0 + q * pq, pq), 1, :],
                bufo.at[slot, pl.ds(q * pq, pq), :],
                insem.at[slot],
            ).start()

    def wait_in(slot):
        pltpu.make_async_copy(
            x_hbm.at[pl.ds(0, _P), 0, :],
            bufe.at[slot],
            insem.at[slot],
        ).wait()
        pltpu.make_async_copy(
            x_hbm.at[pl.ds(0, _P), 1, :],
            bufo.at[slot],
            insem.at[slot],
        ).wait()

    def dma_out(slot, step):
        b = step // (h2 // _P)
        h0 = (step % (h2 // _P)) * _P
        for c in range(4):
            for q in range(_QO):
                pltpu.make_async_copy(
                    obuf.at[slot, c, pl.ds(q * po, po), :],
                    o_hbm.at[b, c, pl.ds(h0 + q * po, po), :],
                    outsem.at[slot],
                ).start()

    def wait_out(slot):
        pltpu.make_async_copy(
            obuf.at[slot],
            o_hbm.at[0, :, pl.ds(0, _P), :],
            outsem.at[slot],
        ).wait()

    def compute(slot):
        e0, e1 = _deinterleave(bufe[slot])
        o0, o1 = _deinterleave(bufo[slot])
        obuf[slot, 0] = e0
        obuf[slot, 1] = e1
        obuf[slot, 2] = o0
        obuf[slot, 3] = o1

    for s0 in range(_SLOTS - 1):
        dma_in(s0, s0)

    def step_fn(s, _):
        slot = jax.lax.rem(s, _SLOTS)
        nxt = jax.lax.rem(s + _SLOTS - 1, _SLOTS)

        @pl.when(s + _SLOTS - 1 < n_steps)
        def _():
            dma_in(nxt, s + _SLOTS - 1)

        wait_in(slot)

        @pl.when(s >= _SLOTS)
        def _():
            wait_out(slot)

        compute(slot)
        dma_out(slot, s)
        return ()

    jax.lax.fori_loop(0, n_steps, step_fn, ())
    for s0 in range(_SLOTS):
        wait_out(jax.lax.rem(n_steps - _SLOTS + s0, _SLOTS))


def kernel(x):
    B, C, H, W = x.shape
    H2, W2 = H // 2, W // 2
    x2 = x.reshape(B * H2, 2, W)
    n_steps = (B * H2) // _P
    body = functools.partial(_body, n_steps=n_steps, h2=H2)
    return pl.pallas_call(
        body,
        in_specs=[pl.BlockSpec(memory_space=pltpu.MemorySpace.HBM)],
        out_specs=pl.BlockSpec(memory_space=pltpu.MemorySpace.HBM),
        out_shape=jax.ShapeDtypeStruct((B, 4 * C, H2, W2), x.dtype),
        scratch_shapes=[
            pltpu.VMEM((_SLOTS, _P, W), x.dtype),
            pltpu.VMEM((_SLOTS, _P, W), x.dtype),
            pltpu.VMEM((_SLOTS, 4, _P, W2), x.dtype),
            pltpu.SemaphoreType.DMA((_SLOTS,)),
            pltpu.SemaphoreType.DMA((_SLOTS,)),
        ],
    )(x2)
